# baseline (speedup 1.0000x reference)
import functools

import jax
import jax.numpy as jnp
from jax import lax
from jax.experimental import pallas as pl
from jax.experimental.pallas import tpu as pltpu
from jax.experimental.pallas import tpu_sc as plsc

N = 65536
D = 64
K = 8192
BN = 1024
NSPLIT = 2
NH = N // NSPLIT

NC = 2
NS = 16
NW = NC * NS
CHUNK = 128
DPAD = 128


def _argmin_body(z_ref, cb_ref, c2_ref, idx_ref, idxf_ref, part_ref):
    z = z_ref[...]
    cb = cb_ref[...]
    dot2 = lax.dot_general(z, cb, (((1,), (1,)), ((), ())),
                           preferred_element_type=jnp.float32)
    z2 = jnp.sum(z * z, axis=1, keepdims=True) * 0.25
    t = z2.reshape(BN // 8, 8, 1) + c2_ref[...][None, :, :]
    d2 = (t - dot2.reshape(BN // 8, 8, K)).reshape(BN, K)
    bm = jnp.min(d2, axis=1, keepdims=True)
    loc = jnp.argmin(d2, axis=1)[:, None]
    idx_ref[...] = loc
    idxf_ref[...] = loc.astype(jnp.float32)
    part_ref[...] = jnp.broadcast_to(jnp.sum(jnp.maximum(bm, 0.0)),
                                     (1, 1, 128))


_distance_argmin = pl.pallas_call(
    _argmin_body,
    grid=(NH // BN,),
    in_specs=[
        pl.BlockSpec((BN, D), lambda i: (i, 0)),
        pl.BlockSpec((K, D), lambda i: (0, 0)),
        pl.BlockSpec((8, K), lambda i: (0, 0)),
    ],
    out_specs=[
        pl.BlockSpec((BN, 1), lambda i: (i, 0)),
        pl.BlockSpec((BN, 1), lambda i: (i, 0)),
        pl.BlockSpec((1, 1, 128), lambda i: (i, 0, 0)),
    ],
    out_shape=[
        jax.ShapeDtypeStruct((NH, 1), jnp.int32),
        jax.ShapeDtypeStruct((NH, 1), jnp.float32),
        jax.ShapeDtypeStruct((NH // BN, 1, 128), jnp.float32),
    ],
)


@functools.cache
def _make_sc_gather(rows):
    chunks_per_w = rows // (NW * CHUNK)
    rows_per_w = chunks_per_w * CHUNK

    @functools.partial(
        pl.kernel,
        out_type=jax.ShapeDtypeStruct((rows, DPAD), jnp.float32),
        mesh=plsc.VectorSubcoreMesh(core_axis_name="c", subcore_axis_name="s"),
        scratch_types=[
            pltpu.VMEM((chunks_per_w, CHUNK), jnp.int32),
            pltpu.VMEM((CHUNK, DPAD), jnp.float32),
            pltpu.VMEM((CHUNK, DPAD), jnp.float32),
            pltpu.SemaphoreType.DMA,
            pltpu.SemaphoreType.DMA,
        ],
    )
    def _sc_gather(idx_hbm, table_hbm, out_hbm, idx_v, rows0, rows1,
                   gsem, ssem):
        wid = lax.axis_index("s") * NC + lax.axis_index("c")
        pltpu.sync_copy(
            idx_hbm.at[pl.ds(wid * chunks_per_w, chunks_per_w)], idx_v)
        base = wid * rows_per_w
        bufs = (rows0, rows1)
        store_cp = None
        for c in range(chunks_per_w):
            buf = bufs[c % 2]
            pltpu.async_copy(table_hbm.at[idx_v.at[c]], buf, gsem).wait()
            if store_cp is not None:
                store_cp.wait()
            store_cp = pltpu.async_copy(
                buf, out_hbm.at[pl.ds(base + c * CHUNK, CHUNK)], ssem)
        store_cp.wait()

    return _sc_gather


def kernel(z_feats, codebook):
    zz = z_feats + z_feats
    c2 = jnp.sum(codebook * codebook, axis=1)
    c2b = jnp.broadcast_to(c2[None, :], (8, K))
    cb_pad = jnp.pad(codebook, ((0, 0), (0, DPAD - D)))
    gather = _make_sc_gather(NH)
    qs, idxfs, loss_sum = [], [], jnp.float32(0.0)
    for h in range(NSPLIT):
        zh = lax.slice_in_dim(zz, h * NH, (h + 1) * NH, axis=0)
        idx_i32, idx_f, parts = _distance_argmin(zh, codebook, c2b)
        loss_sum = loss_sum + jnp.sum(parts[:, 0, 0])
        idx2d = idx_i32.reshape(NH // CHUNK, CHUNK)
        qs.append(gather(idx2d, cb_pad)[:, :D])
        idxfs.append(idx_f)
    loss = loss_sum / jnp.float32(N * D)
    quantized = jnp.concatenate(qs, axis=0) if NSPLIT > 1 else qs[0]
    idx_f_all = jnp.concatenate(idxfs, axis=0) if NSPLIT > 1 else idxfs[0]
    return quantized, loss, loss, idx_f_all


# speedup vs baseline: 1.0213x; 1.0213x over previous
import functools

import jax
import jax.numpy as jnp
from jax import lax
from jax.experimental import pallas as pl
from jax.experimental.pallas import tpu as pltpu
from jax.experimental.pallas import tpu_sc as plsc

N = 65536
D = 64
K = 8192
BN = 1024
NSPLIT = 1
NH = N // NSPLIT

NC = 2
NS = 16
NW = NC * NS
CHUNK = 128
DPAD = 128


def _argmin_body(z_ref, cb_ref, c2_ref, idx_ref, idxf_ref, part_ref):
    z = z_ref[...]
    cb = cb_ref[...]
    dot2 = lax.dot_general(z, cb, (((1,), (1,)), ((), ())),
                           preferred_element_type=jnp.float32)
    z2 = jnp.sum(z * z, axis=1, keepdims=True) * 0.25
    t = z2.reshape(BN // 8, 8, 1) + c2_ref[...][None, :, :]
    d2 = (t - dot2.reshape(BN // 8, 8, K)).reshape(BN, K)
    bm = jnp.min(d2, axis=1, keepdims=True)
    loc = jnp.argmin(d2, axis=1)[:, None]
    idx_ref[...] = loc
    idxf_ref[...] = loc.astype(jnp.float32)
    part_ref[...] = jnp.broadcast_to(jnp.sum(jnp.maximum(bm, 0.0)),
                                     (1, 1, 128))


_distance_argmin = pl.pallas_call(
    _argmin_body,
    grid=(NH // BN,),
    in_specs=[
        pl.BlockSpec((BN, D), lambda i: (i, 0)),
        pl.BlockSpec((K, D), lambda i: (0, 0)),
        pl.BlockSpec((8, K), lambda i: (0, 0)),
    ],
    out_specs=[
        pl.BlockSpec((BN, 1), lambda i: (i, 0)),
        pl.BlockSpec((BN, 1), lambda i: (i, 0)),
        pl.BlockSpec((1, 1, 128), lambda i: (i, 0, 0)),
    ],
    out_shape=[
        jax.ShapeDtypeStruct((NH, 1), jnp.int32),
        jax.ShapeDtypeStruct((NH, 1), jnp.float32),
        jax.ShapeDtypeStruct((NH // BN, 1, 128), jnp.float32),
    ],
)


@functools.cache
def _make_sc_gather(rows):
    chunks_per_w = rows // (NW * CHUNK)
    rows_per_w = chunks_per_w * CHUNK

    @functools.partial(
        pl.kernel,
        out_type=jax.ShapeDtypeStruct((rows, DPAD), jnp.float32),
        mesh=plsc.VectorSubcoreMesh(core_axis_name="c", subcore_axis_name="s"),
        scratch_types=[
            pltpu.VMEM((chunks_per_w, CHUNK), jnp.int32),
            pltpu.VMEM((CHUNK, DPAD), jnp.float32),
            pltpu.VMEM((CHUNK, DPAD), jnp.float32),
            pltpu.SemaphoreType.DMA,
            pltpu.SemaphoreType.DMA,
        ],
    )
    def _sc_gather(idx_hbm, table_hbm, out_hbm, idx_v, rows0, rows1,
                   gsem, ssem):
        wid = lax.axis_index("s") * NC + lax.axis_index("c")
        pltpu.sync_copy(
            idx_hbm.at[pl.ds(wid * chunks_per_w, chunks_per_w)], idx_v)
        base = wid * rows_per_w
        bufs = (rows0, rows1)
        store_cp = None
        for c in range(chunks_per_w):
            buf = bufs[c % 2]
            pltpu.async_copy(table_hbm.at[idx_v.at[c]], buf, gsem).wait()
            if store_cp is not None:
                store_cp.wait()
            store_cp = pltpu.async_copy(
                buf, out_hbm.at[pl.ds(base + c * CHUNK, CHUNK)], ssem)
        store_cp.wait()

    return _sc_gather


def kernel(z_feats, codebook):
    zz = z_feats + z_feats
    c2 = jnp.sum(codebook * codebook, axis=1)
    c2b = jnp.broadcast_to(c2[None, :], (8, K))
    cb_pad = jnp.pad(codebook, ((0, 0), (0, DPAD - D)))
    gather = _make_sc_gather(NH)
    qs, idxfs, loss_sum = [], [], jnp.float32(0.0)
    for h in range(NSPLIT):
        zh = lax.slice_in_dim(zz, h * NH, (h + 1) * NH, axis=0)
        idx_i32, idx_f, parts = _distance_argmin(zh, codebook, c2b)
        loss_sum = loss_sum + jnp.sum(parts[:, 0, 0])
        idx2d = idx_i32.reshape(NH // CHUNK, CHUNK)
        qs.append(gather(idx2d, cb_pad)[:, :D])
        idxfs.append(idx_f)
    loss = loss_sum / jnp.float32(N * D)
    quantized = jnp.concatenate(qs, axis=0) if NSPLIT > 1 else qs[0]
    idx_f_all = jnp.concatenate(idxfs, axis=0) if NSPLIT > 1 else idxfs[0]
    return quantized, loss, loss, idx_f_all
